# baseline (speedup 1.0000x reference)
import functools

import jax
import jax.numpy as jnp
from jax import lax
from jax.experimental import pallas as pl
from jax.experimental.pallas import tpu as pltpu
from jax.experimental.pallas import tpu_sc as plsc

NBUF = 4
LANES = 16
PACK = 4


def _sc_geometry():
    try:
        info = plsc.get_sparse_core_info()
        return info.num_cores, info.num_subcores
    except Exception:
        return 2, 16


def _bcast_lane(vec, lane):
    idx = jnp.full((LANES,), lane, jnp.int32)
    return lax.gather(
        vec,
        idx[:, None],
        lax.GatherDimensionNumbers(
            offset_dims=(),
            collapsed_slice_dims=(0,),
            start_index_map=(0,),
        ),
        slice_sizes=(1,),
        mode=lax.GatherScatterMode.PROMISE_IN_BOUNDS,
    )


def _make_encoder(B, L, NC, NS):
    NW = NC * NS
    assert B % NW == 0
    BPW = B // NW
    assert BPW % NBUF == 0
    RPAD = 64
    inv_l = jnp.float32(1.0 / L)

    mesh = plsc.VectorSubcoreMesh(core_axis_name="c", subcore_axis_name="s")

    @functools.partial(
        pl.kernel,
        out_type=jax.ShapeDtypeStruct((B, 2 * LANES), jnp.float32),
        mesh=mesh,
        scratch_types=[
            pltpu.VMEM((BPW, L), jnp.int32),
            pltpu.VMEM((BPW, RPAD), jnp.int32),
            pltpu.VMEM((NBUF, L, 8 * LANES), jnp.float32),
            pltpu.VMEM((BPW, 2 * LANES), jnp.float32),
        ] + [pltpu.SemaphoreType.DMA] * NBUF,
        compiler_params=pltpu.CompilerParams(
            use_tc_tiling_on_sc=True, needs_layout_passes=False
        ),
    )
    def enc(idx_hbm, off_hbm, table_hbm, out_hbm, idx_v, off_v, rows_v,
            out_v, *sems):
        wid = lax.axis_index("s") * NC + lax.axis_index("c")
        base = wid * BPW

        pltpu.sync_copy(idx_hbm.at[pl.ds(base, BPW)], idx_v)
        pltpu.sync_copy(off_hbm.at[pl.ds(base, BPW)], off_v)

        def start(i, b):
            pltpu.async_copy(table_hbm.at[idx_v.at[i]], rows_v.at[b], sems[b])

        def wait(i, b):
            pltpu.make_async_copy(
                table_hbm.at[idx_v.at[i]], rows_v.at[b], sems[b]
            ).wait()

        for b in range(NBUF):
            start(jnp.int32(b), b)

        iota = lax.iota(jnp.int32, LANES)

        def group(g, carry):
            for b in range(NBUF):
                i = g * NBUF + b
                wait(i, b)
                acc0 = jnp.zeros((LANES,), jnp.float32)
                acc1 = jnp.zeros((LANES,), jnp.float32)
                for j in range(L):
                    ovec = off_v[i, pl.ds((j // LANES) * LANES, LANES)]
                    os_ = _bcast_lane(ovec, j % LANES)
                    js = jnp.full((LANES,), j, jnp.int32)
                    a0 = os_ + iota
                    acc0 = acc0 + plsc.load_gather(rows_v.at[b], [js, a0])
                    acc1 = acc1 + plsc.load_gather(
                        rows_v.at[b], [js, a0 + jnp.int32(LANES)]
                    )
                out_v[i, pl.ds(0, LANES)] = acc0 * inv_l
                out_v[i, pl.ds(LANES, LANES)] = acc1 * inv_l

                @pl.when(g < BPW // NBUF - 1)
                def _():
                    start(i + jnp.int32(NBUF), b)

            return carry

        lax.fori_loop(0, BPW // NBUF, group, jnp.int32(0))

        pltpu.sync_copy(out_v, out_hbm.at[pl.ds(base, BPW)])

    return enc


def kernel(captions, cap_features, w2v_table):
    del cap_features
    B, L = captions.shape
    V, D = w2v_table.shape
    assert D == 2 * LANES and V % PACK == 0
    NC, NS = _sc_geometry()
    table_p = w2v_table.reshape(V // PACK, PACK * D)
    cap = captions.astype(jnp.int32)
    idx_p = cap >> 2
    off = jnp.pad((cap & 3) * D, ((0, 0), (0, 64 - L)))
    enc = _make_encoder(B, L, NC, NS)
    return enc(idx_p, off, table_p)
